# baseline (speedup 1.0000x reference)
import functools

import jax
import jax.numpy as jnp
from jax import lax
from jax.experimental import pallas as pl
from jax.experimental.pallas import tpu as pltpu
from jax.experimental.pallas import tpu_sc as plsc

B = 16384
D = 64
NUM_CORES = 2
NUM_SUBCORES = 16
NW = NUM_CORES * NUM_SUBCORES
BPW = B // NW
L = 16


def _body(uids_hbm, iids_hbm, utab_hbm, itab_hbm,
          score_hbm, uout_hbm, iout_hbm,
          uidx_v, iidx_v, urows_v, irows_v, sc_v,
          sem_u, sem_i, sem_uo, sem_io):
    wid = lax.axis_index("s") * NUM_CORES + lax.axis_index("c")
    base = wid * BPW

    pltpu.sync_copy(uids_hbm.at[pl.ds(base, BPW)], uidx_v)
    pltpu.sync_copy(iids_hbm.at[pl.ds(base, BPW)], iidx_v)

    cu = pltpu.async_copy(utab_hbm.at[uidx_v], urows_v, sem_u)
    ci = pltpu.async_copy(itab_hbm.at[iidx_v], irows_v, sem_i)
    cu.wait()
    ci.wait()

    cuo = pltpu.async_copy(urows_v, uout_hbm.at[pl.ds(base, BPW)], sem_uo)
    cio = pltpu.async_copy(irows_v, iout_hbm.at[pl.ds(base, BPW)], sem_io)

    lane = lax.iota(jnp.int32, L)

    def grp(g, carry):
        rows = lane + g * L
        acc = jnp.zeros((L,), jnp.float32)
        for k in range(D):
            cols = jnp.full((L,), k, jnp.int32)
            u = plsc.load_gather(urows_v, [rows, cols])
            v = plsc.load_gather(irows_v, [rows, cols])
            acc = acc + u * v
        sc_v[pl.ds(g * L, L)] = acc
        return carry

    lax.fori_loop(0, BPW // L, grp, 0)

    pltpu.sync_copy(sc_v, score_hbm.at[pl.ds(base, BPW)])
    cuo.wait()
    cio.wait()


@jax.jit
def _run(user_ids, item_ids, user_table, item_table):
    mesh = plsc.VectorSubcoreMesh(core_axis_name="c", subcore_axis_name="s")
    kern = functools.partial(
        pl.kernel,
        out_type=[
            jax.ShapeDtypeStruct((B,), jnp.float32),
            jax.ShapeDtypeStruct((B, D), jnp.float32),
            jax.ShapeDtypeStruct((B, D), jnp.float32),
        ],
        mesh=mesh,
        compiler_params=pltpu.CompilerParams(
            needs_layout_passes=False, use_tc_tiling_on_sc=False),
        scratch_types=[
            pltpu.VMEM((BPW,), jnp.int32),
            pltpu.VMEM((BPW,), jnp.int32),
            pltpu.VMEM((BPW, D), jnp.float32),
            pltpu.VMEM((BPW, D), jnp.float32),
            pltpu.VMEM((BPW,), jnp.float32),
            pltpu.SemaphoreType.DMA,
            pltpu.SemaphoreType.DMA,
            pltpu.SemaphoreType.DMA,
            pltpu.SemaphoreType.DMA,
        ],
    )(_body)
    return kern(user_ids, item_ids, user_table, item_table)


def kernel(user_ids, item_ids, user_table, item_table):
    score, u_emb, i_emb = _run(
        user_ids.astype(jnp.int32), item_ids.astype(jnp.int32),
        user_table, item_table)
    b = user_ids.shape[0]
    return (score, u_emb.reshape(b, 1, D), i_emb.reshape(b, D, 1))
